# baseline (speedup 1.0000x reference)
import functools

import jax
import jax.numpy as jnp
from jax import lax
from jax.experimental import pallas as pl
from jax.experimental.pallas import tpu as pltpu
from jax.experimental.pallas import tpu_sc as plsc

_NEMB = 32
_BATCH = 16384
_NROW4 = 250000
_NC = 2
_NS = 16
_LANES = 16
_NW = _NC * _NS
_BPW = _BATCH // _NW
_CHUNK = 128
_NCHUNK = _BPW // _CHUNK
_HALF = _BPW // 2


@functools.partial(
    pl.kernel,
    out_type=jax.ShapeDtypeStruct((_BATCH,), jnp.float32),
    mesh=plsc.VectorSubcoreMesh(core_axis_name="c", subcore_axis_name="s"),
    compiler_params=pltpu.CompilerParams(
        needs_layout_passes=False, use_tc_tiling_on_sc=True),
    scratch_types=[
        pltpu.VMEM((_BPW,), jnp.int32),
        pltpu.VMEM((_BPW,), jnp.int32),
        pltpu.VMEM((_BPW,), jnp.int32),
        pltpu.VMEM((_BPW,), jnp.int32),
        pltpu.VMEM((_HALF, 4 * _NEMB), jnp.float32),
        pltpu.VMEM((_HALF, 4 * _NEMB), jnp.float32),
        pltpu.VMEM((_BPW,), jnp.float32),
        pltpu.SemaphoreType.DMA,
    ],
)
def _sc_embed_dot(champ1_hbm, champ2_hbm, w4_hbm, out_hbm,
                  idx1_v, idx2_v, grp1_v, grp2_v, rows1_v, rows2_v, out_v,
                  sem):
    wid = lax.axis_index("s") * _NC + lax.axis_index("c")
    base = wid * _BPW
    pltpu.sync_copy(champ1_hbm.at[pl.ds(base, _BPW)], idx1_v)
    pltpu.sync_copy(champ2_hbm.at[pl.ds(base, _BPW)], idx2_v)

    for v in range(_BPW // _LANES):
        sl = pl.ds(v * _LANES, _LANES)
        grp1_v[sl] = lax.shift_right_logical(idx1_v[sl], 2)
        grp2_v[sl] = lax.shift_right_logical(idx2_v[sl], 2)

    for h in range(_BPW // _HALF):
        h0 = h * _HALF
        copies = []
        for j in range(_HALF // _CHUNK):
            src = pl.ds(h0 + j * _CHUNK, _CHUNK)
            dst = pl.ds(j * _CHUNK, _CHUNK)
            copies.append(
                pltpu.async_copy(w4_hbm.at[grp1_v.at[src]], rows1_v.at[dst],
                                 sem))
            copies.append(
                pltpu.async_copy(w4_hbm.at[grp2_v.at[src]], rows2_v.at[dst],
                                 sem))
        for c in copies:
            c.wait()

        def group_body(g, carry):
            local0 = pl.multiple_of(g * _LANES, _LANES)
            sl = pl.ds(h0 + local0, _LANES)
            rows = local0 + lax.iota(jnp.int32, _LANES)
            off1 = lax.shift_left(jnp.bitwise_and(idx1_v[sl], 3), 5)
            off2 = lax.shift_left(jnp.bitwise_and(idx2_v[sl], 3), 5)
            acc = jnp.zeros((_LANES,), jnp.float32)
            for d in range(_NEMB):
                a = plsc.load_gather(rows1_v, [rows, off1 + d])
                b = plsc.load_gather(rows2_v, [rows, off2 + d])
                acc = acc + a * b
            out_v[pl.ds(h0 + local0, _LANES)] = acc
            return carry

        lax.fori_loop(0, _HALF // _LANES, group_body, 0)
    pltpu.sync_copy(out_v, out_hbm.at[pl.ds(base, _BPW)])


def kernel(champ1, champ2, W):
    w4 = W.reshape(_NROW4, 4 * _NEMB)
    out = _sc_embed_dot(champ1.astype(jnp.int32), champ2.astype(jnp.int32),
                        w4)
    return out.reshape(_BATCH, 1, 1)
